# baseline (speedup 1.0000x reference)
import functools

import jax
import jax.numpy as jnp
from jax import lax
from jax.experimental import pallas as pl
from jax.experimental.pallas import tpu as pltpu
from jax.experimental.pallas import tpu_sc as plsc

NC = 2
NS = 16
CH = 128

_MESH = dict(core_axis_name="c", subcore_axis_name="s")


def _row_split(n):
    rpt = (-(-n // NS) + 7) // 8 * 8
    last = n - rpt * (NS - 1)
    assert last > 0 and last % 8 == 0 and rpt % 8 == 0
    return rpt, last


@functools.lru_cache(maxsize=None)
def _deg_call(n, ep):
    nchunks = ep // (NC * NS * CH)
    assert nchunks % 4 == 0
    rpt, last = _row_split(n)
    mesh = plsc.VectorSubcoreMesh(**_MESH)
    zlen = -(-rpt // 16) * 16

    @functools.partial(
        pl.kernel,
        out_type=jax.ShapeDtypeStruct((NC * n,), jnp.float32),
        mesh=mesh,
        scratch_types=[
            pltpu.VMEM((nchunks, CH), jnp.int32),
            pltpu.VMEM((nchunks, CH), jnp.float32),
            pltpu.VMEM((zlen,), jnp.float32),
            pltpu.VMEM_SHARED((rpt * NS,), jnp.float32),
            pltpu.SemaphoreType.DMA,
        ],
        compiler_params=pltpu.CompilerParams(use_tc_tiling_on_sc=False),
    )
    def deg_kernel(dst_hbm, w_hbm, out_hbm, dst_all, w_all, zbuf, deg_sh, ssem):
        cid = lax.axis_index("c")
        sid = lax.axis_index("s")
        off = sid * rpt

        def zv(i, c2):
            zbuf[pl.ds(i * 16, 16)] = jnp.zeros((16,), jnp.float32)
            return c2

        lax.fori_loop(0, zlen // 16, zv, 0)

        @pl.when(sid < NS - 1)
        def _():
            pltpu.sync_copy(zbuf.at[pl.ds(0, rpt)], deg_sh.at[pl.ds(off, rpt)])

        @pl.when(sid == NS - 1)
        def _():
            pltpu.sync_copy(zbuf.at[pl.ds(0, last)], deg_sh.at[pl.ds(off, last)])

        cbase = (cid * NS + sid) * nchunks
        pltpu.sync_copy(dst_hbm.at[pl.ds(cbase, nchunks)], dst_all)
        pltpu.sync_copy(w_hbm.at[pl.ds(cbase, nchunks)], w_all)
        plsc.subcore_barrier()

        FD = 4

        def group(g, carry):
            for j in range(FD):
                k = g * FD + j
                pltpu.async_copy(w_all.at[k], deg_sh.at[dst_all.at[k]], ssem, add=True)
            for j in range(FD):
                pltpu.make_async_copy(w_all.at[0], deg_sh.at[dst_all.at[0]], ssem).wait()
            return carry

        lax.fori_loop(0, nchunks // FD, group, 0)
        plsc.subcore_barrier()

        def wout(total):
            pltpu.sync_copy(deg_sh.at[pl.ds(off, total)], zbuf.at[pl.ds(0, total)])
            pltpu.sync_copy(zbuf.at[pl.ds(0, total)], out_hbm.at[pl.ds(cid * n + off, total)])

        @pl.when(sid < NS - 1)
        def _():
            wout(rpt)

        @pl.when(sid == NS - 1)
        def _():
            wout(last)

    return deg_kernel


@functools.lru_cache(maxsize=None)
def _edge_call(n, d, ep):
    d2 = d // NC
    nchunks = ep // (NS * CH)
    NB = 3
    assert nchunks % NB == 0
    rpt, last = _row_split(n)
    mesh = plsc.VectorSubcoreMesh(**_MESH)

    @functools.partial(
        pl.kernel,
        out_type=jax.ShapeDtypeStruct((NC, n, d2), jnp.float32),
        mesh=mesh,
        scratch_types=[
            pltpu.VMEM((nchunks, CH), jnp.int32),
            pltpu.VMEM((nchunks, CH), jnp.int32),
            pltpu.VMEM((nchunks, CH), jnp.float32),
            pltpu.VMEM((NB, CH, d2), jnp.float32),
            pltpu.VMEM_SHARED((rpt * NS, d2), jnp.float32),
            pltpu.SemaphoreType.DMA,
            pltpu.SemaphoreType.DMA,
            pltpu.SemaphoreType.DMA,
        ],
        compiler_params=pltpu.CompilerParams(use_tc_tiling_on_sc=False),
    )
    def edge_kernel(y_hbm, src_hbm, dst_hbm, w_hbm, out_hbm,
                    src_all, dst_all, w_all, rows, acc_sh, gsem0, gsem1, gsem2):
        gsems = (gsem0, gsem1, gsem2)
        cid = lax.axis_index("c")
        sid = lax.axis_index("s")
        off = sid * rpt
        cbase = sid * nchunks

        pltpu.sync_copy(src_hbm.at[pl.ds(cbase, nchunks)], src_all)
        pltpu.sync_copy(dst_hbm.at[pl.ds(cbase, nchunks)], dst_all)
        pltpu.sync_copy(w_hbm.at[pl.ds(cbase, nchunks)], w_all)

        ybase = cid * n

        def adj(k, c2):
            for g in range(CH // 16):
                sl = pl.ds(g * 16, 16)
                src_all[k, sl] = src_all[k, sl] + ybase
            return c2

        lax.fori_loop(0, nchunks, adj, 0)

        def issue_gather(k_, buf):
            pltpu.async_copy(y_hbm.at[src_all.at[k_]], rows.at[buf], gsems[buf])

        def zrow(r, c2):
            for f in range(d2 // 16):
                rows[NB - 1, r, pl.ds(f * 16, 16)] = jnp.zeros((16,), jnp.float32)
            return c2

        lax.fori_loop(0, CH, zrow, 0)

        def init_acc(total):
            done = 0
            while done < total:
                size = min(CH, total - done)
                pltpu.sync_copy(rows.at[NB - 1, pl.ds(0, size)],
                                acc_sh.at[pl.ds(off + done, size)])
                done += size

        @pl.when(sid < NS - 1)
        def _():
            init_acc(rpt)

        @pl.when(sid == NS - 1)
        def _():
            init_acc(last)

        plsc.subcore_barrier()
        issue_gather(0, 0)
        issue_gather(1, 1)

        def group(g, carry):
            for b in range(NB):
                k = NB * g + b
                pltpu.make_async_copy(y_hbm.at[src_all.at[k]], rows.at[b],
                                      gsems[b]).wait()
                bp = (b + NB - 1) % NB

                @pl.when(k + NB - 1 < nchunks)
                def _():
                    issue_gather(k + NB - 1, bp)

                def edge16(g16, c2):
                    w16 = w_all[k, pl.ds(g16 * 16, 16)]
                    for j in range(16):
                        ws = w16[j]
                        i = g16 * 16 + j
                        for f in range(d2 // 16):
                            sl = pl.ds(f * 16, 16)
                            rows[b, i, sl] = rows[b, i, sl] * ws
                    return c2

                lax.fori_loop(0, CH // 16, edge16, 0, unroll=True)
                pltpu.sync_copy(rows.at[b], acc_sh.at[dst_all.at[k]], add=True)
            return carry

        lax.fori_loop(0, nchunks // NB, group, 0)
        plsc.subcore_barrier()

        def wout(total):
            done = 0
            while done < total:
                size = min(CH, total - done)
                pltpu.sync_copy(acc_sh.at[pl.ds(off + done, size)], rows.at[0, pl.ds(0, size)])
                pltpu.sync_copy(rows.at[0, pl.ds(0, size)], out_hbm.at[cid, pl.ds(off + done, size)])
                done += size

        @pl.when(sid < NS - 1)
        def _():
            wout(rpt)

        @pl.when(sid == NS - 1)
        def _():
            wout(last)

    return edge_kernel


_R = 2000


@functools.lru_cache(maxsize=None)
def _prep_call(n, d):
    d2 = d // NC

    def body(degp_ref, x_ref, w_ref, y_ref, dis_ref):
        deg = jnp.sum(degp_ref[...], axis=1, keepdims=True) + 1.0
        dis = jnp.where(deg > 0.0, lax.rsqrt(deg), 0.0)
        xw = jnp.dot(x_ref[...], w_ref[...], preferred_element_type=jnp.float32)
        y = xw * dis
        for c in range(NC):
            y_ref[c] = y[:, c * d2:(c + 1) * d2]
        dis_ref[...] = dis

    return pl.pallas_call(
        body,
        grid=(n // _R,),
        in_specs=[
            pl.BlockSpec((_R, NC), lambda i: (i, 0)),
            pl.BlockSpec((_R, d), lambda i: (i, 0)),
            pl.BlockSpec((d, d), lambda i: (0, 0)),
        ],
        out_specs=[
            pl.BlockSpec((NC, _R, d2), lambda i: (0, i, 0)),
            pl.BlockSpec((_R, 1), lambda i: (i, 0)),
        ],
        out_shape=[
            jax.ShapeDtypeStruct((NC, n, d2), jnp.float32),
            jax.ShapeDtypeStruct((n, 1), jnp.float32),
        ],
    )


@functools.lru_cache(maxsize=None)
def _mid_call(n, d):
    d2 = d // NC

    def body(accs_ref, y_ref, dis_ref, b_ref, w_ref, y2_ref):
        acc = jnp.concatenate([accs_ref[c] + y_ref[c] for c in range(NC)], axis=-1)
        dis = dis_ref[...]
        h = jnp.maximum(acc * dis + b_ref[...], 0.0)
        y2 = jnp.dot(h, w_ref[...], preferred_element_type=jnp.float32) * dis
        for c in range(NC):
            y2_ref[c] = y2[:, c * d2:(c + 1) * d2]

    return pl.pallas_call(
        body,
        grid=(n // _R,),
        in_specs=[
            pl.BlockSpec((NC, _R, d2), lambda i: (0, i, 0)),
            pl.BlockSpec((NC, _R, d2), lambda i: (0, i, 0)),
            pl.BlockSpec((_R, 1), lambda i: (i, 0)),
            pl.BlockSpec((1, d), lambda i: (0, 0)),
            pl.BlockSpec((d, d), lambda i: (0, 0)),
        ],
        out_specs=pl.BlockSpec((NC, _R, d2), lambda i: (0, i, 0)),
        out_shape=jax.ShapeDtypeStruct((NC, n, d2), jnp.float32),
    )


@functools.lru_cache(maxsize=None)
def _post_call(n, d):
    d2 = d // NC

    def body(accs_ref, y2_ref, dis_ref, b_ref, out_ref):
        acc = jnp.concatenate([accs_ref[c] + y2_ref[c] for c in range(NC)], axis=-1)
        out_ref[...] = jnp.maximum(acc * dis_ref[...] + b_ref[...], 0.0)

    return pl.pallas_call(
        body,
        grid=(n // _R,),
        in_specs=[
            pl.BlockSpec((NC, _R, d2), lambda i: (0, i, 0)),
            pl.BlockSpec((NC, _R, d2), lambda i: (0, i, 0)),
            pl.BlockSpec((_R, 1), lambda i: (i, 0)),
            pl.BlockSpec((1, d), lambda i: (0, 0)),
        ],
        out_specs=pl.BlockSpec((_R, d), lambda i: (i, 0)),
        out_shape=jax.ShapeDtypeStruct((n, d), jnp.float32),
    )


def kernel(x, edge_index, weight, W1, b1, W2, b2):
    n, d = x.shape
    e = edge_index.shape[1]
    tile_e = NC * NS * CH
    nch = (e + tile_e - 1) // tile_e
    nch = (nch + 11) // 12 * 12
    ep = nch * tile_e
    pad = ep - e
    src_p = jnp.pad(edge_index[0], (0, pad)).reshape(-1, CH)
    dst_p = jnp.pad(edge_index[1], (0, pad)).reshape(-1, CH)
    w_p = jnp.pad(weight, (0, pad)).reshape(-1, CH)

    d2 = d // NC
    degp = _deg_call(n, ep)(dst_p, w_p).reshape(NC, n)
    y1, dis = _prep_call(n, d)(degp.T, x, W1)
    accs1 = _edge_call(n, d, ep)(y1.reshape(NC * n, d2), src_p, dst_p, w_p)
    y2 = _mid_call(n, d)(accs1, y1, dis, b1.reshape(1, d), W2)
    accs2 = _edge_call(n, d, ep)(y2.reshape(NC * n, d2), src_p, dst_p, w_p)
    out = _post_call(n, d)(accs2, y2, dis, b2.reshape(1, d))
    return out


# speedup vs baseline: 1.9897x; 1.9897x over previous
import functools

import jax
import jax.numpy as jnp
from jax import lax
from jax.experimental import pallas as pl
from jax.experimental.pallas import tpu as pltpu
from jax.experimental.pallas import tpu_sc as plsc

NC = 2
NS = 16
CH = 128

_MESH = dict(core_axis_name="c", subcore_axis_name="s")


def _row_split(n):
    rpt = (-(-n // NS) + 7) // 8 * 8
    last = n - rpt * (NS - 1)
    assert last > 0 and last % 8 == 0 and rpt % 8 == 0
    return rpt, last


@functools.lru_cache(maxsize=None)
def _deg_call(n, ep):
    nchunks = ep // (NC * NS * CH)
    assert nchunks % 8 == 0
    rpt, last = _row_split(n)
    mesh = plsc.VectorSubcoreMesh(**_MESH)
    zlen = -(-rpt // 16) * 16

    @functools.partial(
        pl.kernel,
        out_type=jax.ShapeDtypeStruct((NC * n,), jnp.float32),
        mesh=mesh,
        scratch_types=[
            pltpu.VMEM((nchunks, CH), jnp.int32),
            pltpu.VMEM((nchunks, CH), jnp.float32),
            pltpu.VMEM((zlen,), jnp.float32),
            pltpu.VMEM_SHARED((rpt * NS,), jnp.float32),
            pltpu.SemaphoreType.DMA,
        ],
        compiler_params=pltpu.CompilerParams(use_tc_tiling_on_sc=False),
    )
    def deg_kernel(dst_hbm, w_hbm, out_hbm, dst_all, w_all, zbuf, deg_sh, ssem):
        cid = lax.axis_index("c")
        sid = lax.axis_index("s")
        off = sid * rpt

        def zv(i, c2):
            zbuf[pl.ds(i * 16, 16)] = jnp.zeros((16,), jnp.float32)
            return c2

        lax.fori_loop(0, zlen // 16, zv, 0)

        @pl.when(sid < NS - 1)
        def _():
            pltpu.sync_copy(zbuf.at[pl.ds(0, rpt)], deg_sh.at[pl.ds(off, rpt)])

        @pl.when(sid == NS - 1)
        def _():
            pltpu.sync_copy(zbuf.at[pl.ds(0, last)], deg_sh.at[pl.ds(off, last)])

        cbase = (cid * NS + sid) * nchunks
        pltpu.sync_copy(dst_hbm.at[pl.ds(cbase, nchunks)], dst_all)
        pltpu.sync_copy(w_hbm.at[pl.ds(cbase, nchunks)], w_all)
        plsc.subcore_barrier()

        FD = 8

        def group(g, carry):
            for j in range(FD):
                k = g * FD + j
                pltpu.async_copy(w_all.at[k], deg_sh.at[dst_all.at[k]], ssem, add=True)
            for j in range(FD):
                pltpu.make_async_copy(w_all.at[0], deg_sh.at[dst_all.at[0]], ssem).wait()
            return carry

        lax.fori_loop(0, nchunks // FD, group, 0)
        plsc.subcore_barrier()

        def wout(total):
            pltpu.sync_copy(deg_sh.at[pl.ds(off, total)], zbuf.at[pl.ds(0, total)])
            pltpu.sync_copy(zbuf.at[pl.ds(0, total)], out_hbm.at[pl.ds(cid * n + off, total)])

        @pl.when(sid < NS - 1)
        def _():
            wout(rpt)

        @pl.when(sid == NS - 1)
        def _():
            wout(last)

    return deg_kernel


@functools.lru_cache(maxsize=None)
def _edge_call(n, d, ep):
    d2 = d // NC
    nchunks = ep // (NS * CH)
    NB = 2
    assert nchunks % NB == 0
    rpt, last = _row_split(n)
    mesh = plsc.VectorSubcoreMesh(**_MESH)

    @functools.partial(
        pl.kernel,
        out_type=jax.ShapeDtypeStruct((NC, n, d2), jnp.float32),
        mesh=mesh,
        scratch_types=[
            pltpu.VMEM((nchunks, CH), jnp.int32),
            pltpu.VMEM((nchunks, CH), jnp.int32),
            pltpu.VMEM((nchunks, CH), jnp.float32),
            pltpu.VMEM((NB, CH, d2), jnp.float32),
            pltpu.VMEM_SHARED((rpt * NS, d2), jnp.float32),
            pltpu.SemaphoreType.DMA,
            pltpu.SemaphoreType.DMA,
        ],
        compiler_params=pltpu.CompilerParams(use_tc_tiling_on_sc=False),
    )
    def edge_kernel(y_hbm, src_hbm, dst_hbm, w_hbm, out_hbm,
                    src_all, dst_all, w_all, rows, acc_sh, gsem0, gsem1):
        gsems = (gsem0, gsem1)
        cid = lax.axis_index("c")
        sid = lax.axis_index("s")
        off = sid * rpt
        cbase = sid * nchunks

        pltpu.sync_copy(src_hbm.at[pl.ds(cbase, nchunks)], src_all)
        pltpu.sync_copy(dst_hbm.at[pl.ds(cbase, nchunks)], dst_all)
        pltpu.sync_copy(w_hbm.at[pl.ds(cbase, nchunks)], w_all)

        ybase = cid * n

        def adj(k, c2):
            for g in range(CH // 16):
                sl = pl.ds(g * 16, 16)
                src_all[k, sl] = src_all[k, sl] + ybase
            return c2

        lax.fori_loop(0, nchunks, adj, 0)

        def issue_gather(k_, buf):
            pltpu.async_copy(y_hbm.at[src_all.at[k_]], rows.at[buf], gsems[buf])

        def zrow(r, c2):
            for f in range(d2 // 16):
                rows[NB - 1, r, pl.ds(f * 16, 16)] = jnp.zeros((16,), jnp.float32)
            return c2

        lax.fori_loop(0, CH, zrow, 0)

        def init_acc(total):
            done = 0
            while done < total:
                size = min(CH, total - done)
                pltpu.sync_copy(rows.at[NB - 1, pl.ds(0, size)],
                                acc_sh.at[pl.ds(off + done, size)])
                done += size

        @pl.when(sid < NS - 1)
        def _():
            init_acc(rpt)

        @pl.when(sid == NS - 1)
        def _():
            init_acc(last)

        plsc.subcore_barrier()
        issue_gather(0, 0)

        def group(g, carry):
            for b in range(NB):
                k = NB * g + b
                pltpu.make_async_copy(y_hbm.at[src_all.at[k]], rows.at[b],
                                      gsems[b]).wait()
                bp = (b + NB - 1) % NB

                @pl.when(k + NB - 1 < nchunks)
                def _():
                    issue_gather(k + NB - 1, bp)

                def edge16(g16, c2):
                    w16 = w_all[k, pl.ds(g16 * 16, 16)]
                    for j in range(16):
                        ws = w16[j]
                        i = g16 * 16 + j
                        for f in range(d2 // 16):
                            sl = pl.ds(f * 16, 16)
                            rows[b, i, sl] = rows[b, i, sl] * ws
                    return c2

                lax.fori_loop(0, CH // 16, edge16, 0, unroll=True)
                pltpu.sync_copy(rows.at[b], acc_sh.at[dst_all.at[k]], add=True)
            return carry

        lax.fori_loop(0, nchunks // NB, group, 0)
        plsc.subcore_barrier()

        def wout(total):
            done = 0
            while done < total:
                size = min(CH, total - done)
                pltpu.sync_copy(acc_sh.at[pl.ds(off + done, size)], rows.at[0, pl.ds(0, size)])
                pltpu.sync_copy(rows.at[0, pl.ds(0, size)], out_hbm.at[cid, pl.ds(off + done, size)])
                done += size

        @pl.when(sid < NS - 1)
        def _():
            wout(rpt)

        @pl.when(sid == NS - 1)
        def _():
            wout(last)

    return edge_kernel


_R = 2000


@functools.lru_cache(maxsize=None)
def _prep_call(n, d):
    d2 = d // NC

    def body(degp_ref, x_ref, w_ref, y_ref, dis_ref):
        deg = jnp.sum(degp_ref[...], axis=1, keepdims=True) + 1.0
        dis = jnp.where(deg > 0.0, lax.rsqrt(deg), 0.0)
        xw = jnp.dot(x_ref[...], w_ref[...], preferred_element_type=jnp.float32)
        y = xw * dis
        for c in range(NC):
            y_ref[c] = y[:, c * d2:(c + 1) * d2]
        dis_ref[...] = dis

    return pl.pallas_call(
        body,
        grid=(n // _R,),
        in_specs=[
            pl.BlockSpec((_R, NC), lambda i: (i, 0)),
            pl.BlockSpec((_R, d), lambda i: (i, 0)),
            pl.BlockSpec((d, d), lambda i: (0, 0)),
        ],
        out_specs=[
            pl.BlockSpec((NC, _R, d2), lambda i: (0, i, 0)),
            pl.BlockSpec((_R, 1), lambda i: (i, 0)),
        ],
        out_shape=[
            jax.ShapeDtypeStruct((NC, n, d2), jnp.float32),
            jax.ShapeDtypeStruct((n, 1), jnp.float32),
        ],
    )


@functools.lru_cache(maxsize=None)
def _mid_call(n, d):
    d2 = d // NC

    def body(accs_ref, y_ref, dis_ref, b_ref, w_ref, y2_ref):
        acc = jnp.concatenate([accs_ref[c] + y_ref[c] for c in range(NC)], axis=-1)
        dis = dis_ref[...]
        h = jnp.maximum(acc * dis + b_ref[...], 0.0)
        y2 = jnp.dot(h, w_ref[...], preferred_element_type=jnp.float32) * dis
        for c in range(NC):
            y2_ref[c] = y2[:, c * d2:(c + 1) * d2]

    return pl.pallas_call(
        body,
        grid=(n // _R,),
        in_specs=[
            pl.BlockSpec((NC, _R, d2), lambda i: (0, i, 0)),
            pl.BlockSpec((NC, _R, d2), lambda i: (0, i, 0)),
            pl.BlockSpec((_R, 1), lambda i: (i, 0)),
            pl.BlockSpec((1, d), lambda i: (0, 0)),
            pl.BlockSpec((d, d), lambda i: (0, 0)),
        ],
        out_specs=pl.BlockSpec((NC, _R, d2), lambda i: (0, i, 0)),
        out_shape=jax.ShapeDtypeStruct((NC, n, d2), jnp.float32),
    )


@functools.lru_cache(maxsize=None)
def _post_call(n, d):
    d2 = d // NC

    def body(accs_ref, y2_ref, dis_ref, b_ref, out_ref):
        acc = jnp.concatenate([accs_ref[c] + y2_ref[c] for c in range(NC)], axis=-1)
        out_ref[...] = jnp.maximum(acc * dis_ref[...] + b_ref[...], 0.0)

    return pl.pallas_call(
        body,
        grid=(n // _R,),
        in_specs=[
            pl.BlockSpec((NC, _R, d2), lambda i: (0, i, 0)),
            pl.BlockSpec((NC, _R, d2), lambda i: (0, i, 0)),
            pl.BlockSpec((_R, 1), lambda i: (i, 0)),
            pl.BlockSpec((1, d), lambda i: (0, 0)),
        ],
        out_specs=pl.BlockSpec((_R, d), lambda i: (i, 0)),
        out_shape=jax.ShapeDtypeStruct((n, d), jnp.float32),
    )


def kernel(x, edge_index, weight, W1, b1, W2, b2):
    n, d = x.shape
    e = edge_index.shape[1]
    tile_e = NC * NS * CH
    nch = (e + tile_e - 1) // tile_e
    nch = (nch + 7) // 8 * 8
    ep = nch * tile_e
    pad = ep - e
    src_p = jnp.pad(edge_index[0], (0, pad)).reshape(-1, CH)
    dst_p = jnp.pad(edge_index[1], (0, pad)).reshape(-1, CH)
    w_p = jnp.pad(weight, (0, pad)).reshape(-1, CH)

    d2 = d // NC
    degp = _deg_call(n, ep)(dst_p, w_p).reshape(NC, n)
    y1, dis = _prep_call(n, d)(degp.T, x, W1)
    accs1 = _edge_call(n, d, ep)(y1.reshape(NC * n, d2), src_p, dst_p, w_p)
    y2 = _mid_call(n, d)(accs1, y1, dis, b1.reshape(1, d), W2)
    accs2 = _edge_call(n, d, ep)(y2.reshape(NC * n, d2), src_p, dst_p, w_p)
    out = _post_call(n, d)(accs2, y2, dis, b2.reshape(1, d))
    return out
